# baseline (speedup 1.0000x reference)
import functools

import jax
import jax.numpy as jnp
import numpy as np
from jax import lax
from jax.experimental import pallas as pl
from jax.experimental.pallas import tpu as pltpu
from jax.experimental.pallas import tpu_sc as plsc

N_LEVELS = 16
F_PER_LEVEL = 2
LOG2_T = 19
T = 1 << LOG2_T
MASK = T - 1
BASE_RES = 16
PER_LEVEL_SCALE = 1.5
IMG_RES = 512
OUT_FEATURES = 4
N = IMG_RES * IMG_RES

HASH_K = np.int32(-1640531535)

RES_NP = np.array(
    [float(np.floor(BASE_RES * PER_LEVEL_SCALE**l)) for l in range(N_LEVELS)],
    dtype=np.float32,
)

NC = 2
NS = 16
NW = NC * NS
C = 128
G = C // 16
N_W = N // NW
N_CHUNKS = N_W // C

W_ROW = 16
R = 4

N_DENSE = 6
DENSE_NL = [int(RES_NP[l]) + 1 for l in range(N_DENSE)]
DENSE_SZ = [nl * nl for nl in DENSE_NL]
DENSE_OFF = [sum(DENSE_SZ[:l]) for l in range(N_DENSE)]
DENSE_TOT = sum(DENSE_SZ)
DENSE_NB = [(sz + C - 1) // C for sz in DENSE_SZ]
N_STREAM = N_LEVELS - N_DENSE
N_IT = N_CHUNKS * N_STREAM

TAB_WORDS = N_LEVELS * T * F_PER_LEVEL
TABN_ROWS = TAB_WORDS // 128
ROWS_PER_TILE = TABN_ROWS // NW
IL_CHUNK = 64

_SC_PARAMS = pltpu.CompilerParams(
    needs_layout_passes=False, use_tc_tiling_on_sc=False)
_SC_MESH = plsc.VectorSubcoreMesh(core_axis_name="c", subcore_axis_name="s")


IL_N = ROWS_PER_TILE // IL_CHUNK


def _interleave_body(tabn, tabi, inbuf, outbuf, semi, semo):
    wid = lax.axis_index("s") * NC + lax.axis_index("c")
    r0 = wid * ROWS_PER_TILE
    lane = lax.iota(jnp.int32, 16)

    def fire_in(i, s):
        pltpu.async_copy(tabn.at[pl.ds(r0 + i * IL_CHUNK, IL_CHUNK)],
                         inbuf.at[s], semi)

    for s in range(2):
        fire_in(s, s)

    def chunk_body(i, _):
        for s in range(2):
            @pl.when(i % 2 == s)
            def _(s=s):
                pltpu.make_async_copy(tabn.at[pl.ds(0, IL_CHUNK)],
                                      inbuf.at[s], semi).wait()

                @pl.when(i >= 2)
                def _():
                    pltpu.make_async_copy(
                        outbuf.at[s],
                        tabi.at[pl.ds(0, IL_CHUNK * 128)], semo).wait()
                for p in range(IL_CHUNK // 2):
                    for v in range(8):
                        f0 = inbuf[s, 2 * p, pl.ds(v * 16, 16)]
                        f1 = inbuf[s, 2 * p + 1, pl.ds(v * 16, 16)]
                        w = lane * 2 + (p * 256 + v * 32)
                        plsc.store_scatter(outbuf.at[s], [w], f0)
                        plsc.store_scatter(outbuf.at[s], [w + 1], f1)
                rbase = r0 + i * IL_CHUNK
                pltpu.async_copy(
                    outbuf.at[s], tabi.at[pl.ds(rbase * 128, IL_CHUNK * 128)],
                    semo)

                @pl.when(i + 2 < IL_N)
                def _():
                    fire_in(i + 2, s)
        return ()

    lax.fori_loop(0, IL_N, chunk_body, ())
    for s in range(2):
        pltpu.make_async_copy(outbuf.at[s],
                              tabi.at[pl.ds(0, IL_CHUNK * 128)], semo).wait()


_interleave = functools.partial(
    pl.kernel,
    mesh=_SC_MESH,
    out_type=jax.ShapeDtypeStruct((TAB_WORDS,), jnp.float32),
    compiler_params=_SC_PARAMS,
    scratch_types=[
        pltpu.VMEM((2, IL_CHUNK, 128), jnp.float32),
        pltpu.VMEM((2, IL_CHUNK * 128), jnp.float32),
        pltpu.SemaphoreType.DMA,
        pltpu.SemaphoreType.DMA,
    ],
)(_interleave_body)


def _encode_body(xyv, tab, res, enc,
                 xyall, wxbuf, wybuf, idxb, pbuf, rows, encbuf, resb, dense,
                 *sems):
    wid = lax.axis_index("s") * NC + lax.axis_index("c")
    base_pt = wid * N_W
    pltpu.sync_copy(res, resb)
    pltpu.sync_copy(xyv.at[pl.ds(base_pt * 2, N_W * 2)], xyall)
    lane = lax.iota(jnp.int32, 16)
    zeros16 = jnp.zeros((16,), jnp.int32)

    for lvl in range(N_DENSE):
        nl = DENSE_NL[lvl]
        emax = DENSE_SZ[lvl] - 1
        offw = DENSE_OFF[lvl] * 2
        lbase = lvl * T
        nb = DENSE_NB[lvl]

        def fire_b(b, s, nl=nl, emax=emax, lbase=lbase):
            for g in range(G):
                sl = pl.ds(g * 16, 16)
                e = jnp.minimum(lane + (b * C + g * 16), emax)
                cx = e // nl
                cy = e - cx * nl
                h = (cx ^ (cy * HASH_K)) & MASK
                idxb[s, 0, sl] = (lbase + h) >> 3
                pbuf[s, 0, sl] = (h & 7) * 2
            pltpu.async_copy(tab.at[idxb.at[s, 0]], rows.at[s, 0], sems[s])

        def extract_b(b, s, nl=nl, emax=emax, offw=offw):
            pltpu.make_async_copy(tab.at[pl.ds(0, C)], rows.at[s, 0],
                                  sems[s]).wait()
            sf = jnp.full((16,), s, jnp.int32)
            for g in range(G):
                sl = pl.ds(g * 16, 16)
                e = jnp.minimum(lane + (b * C + g * 16), emax)
                d = offw + e * 2
                p = pbuf[s, 0, sl]
                pt = lane + g * 16
                f0 = plsc.load_gather(rows, [sf, zeros16, pt, p])
                f1 = plsc.load_gather(rows, [sf, zeros16, pt, p + 1])
                plsc.store_scatter(dense, [d], f0)
                plsc.store_scatter(dense, [d + 1], f1)

        def build_body(b, _, fire_b=fire_b, extract_b=extract_b, nb=nb):
            for s in range(2):
                @pl.when(b % 2 == s)
                def _(s=s):
                    @pl.when(b >= 2)
                    def _():
                        extract_b(b - 2, s)

                    @pl.when(b < nb)
                    def _():
                        fire_b(b, s)
            return ()

        lax.fori_loop(0, nb + 2, build_body, ())

    def fire(it, s, sem):
        ch = it // N_STREAM
        l = N_DENSE + it % N_STREAM
        resv = plsc.load_gather(resb, [jnp.full((16,), l, jnp.int32)])
        lbase = l * T
        for g in range(G):
            sl = pl.ds(g * 16, 16)
            px = xyall[pl.ds(ch * 256 + g * 16, 16)] * resv
            py = xyall[pl.ds(ch * 256 + 128 + g * 16, 16)] * resv
            ix = px.astype(jnp.int32)
            iy = py.astype(jnp.int32)
            wxbuf[s, sl] = px - ix.astype(jnp.float32)
            wybuf[s, sl] = py - iy.astype(jnp.float32)
            hy0 = iy * HASH_K
            hy1 = (iy + 1) * HASH_K
            ix1 = ix + 1
            for c, h in ((0, (ix ^ hy0) & MASK), (1, (ix1 ^ hy0) & MASK),
                         (2, (ix ^ hy1) & MASK), (3, (ix1 ^ hy1) & MASK)):
                idxb[s, c, sl] = (lbase + h) >> 3
                pbuf[s, c, sl] = (h & 7) * 2
        for c in range(4):
            pltpu.async_copy(tab.at[idxb.at[s, c]], rows.at[s, c], sem)

    def combine(it, s, sem):
        l = N_DENSE + it % N_STREAM
        for c in range(4):
            pltpu.make_async_copy(tab.at[pl.ds(0, C)], rows.at[s, c],
                                  sem).wait()
        sf = jnp.full((16,), s, jnp.int32)
        f_lo = jnp.full((16,), 2 * l, jnp.int32)
        f_hi = f_lo + 1
        for g in range(G):
            sl = pl.ds(g * 16, 16)
            wx = wxbuf[s, sl]
            wy = wybuf[s, sl]
            ex = 1.0 - wx
            ey = 1.0 - wy
            pt = lane + g * 16
            acc0 = jnp.zeros((16,), jnp.float32)
            acc1 = jnp.zeros((16,), jnp.float32)
            for c, wgt in ((0, ex * ey), (1, wx * ey),
                           (2, ex * wy), (3, wx * wy)):
                cf = jnp.full((16,), c, jnp.int32)
                p = pbuf[s, c, sl]
                f0 = plsc.load_gather(rows, [sf, cf, pt, p])
                f1 = plsc.load_gather(rows, [sf, cf, pt, p + 1])
                acc0 = acc0 + f0 * wgt
                acc1 = acc1 + f1 * wgt
            plsc.store_scatter(encbuf, [pt, f_lo], acc0)
            plsc.store_scatter(encbuf, [pt, f_hi], acc1)

    def coarse_fill(ch):
        for lvl in range(N_DENSE):
            nl = DENSE_NL[lvl]
            offw = DENSE_OFF[lvl] * 2
            resv = jnp.full((16,), float(RES_NP[lvl]), jnp.float32)
            f_lo = jnp.full((16,), 2 * lvl, jnp.int32)
            f_hi = f_lo + 1
            for g in range(G):
                px = xyall[pl.ds(ch * 256 + g * 16, 16)] * resv
                py = xyall[pl.ds(ch * 256 + 128 + g * 16, 16)] * resv
                ix = px.astype(jnp.int32)
                iy = py.astype(jnp.int32)
                wx = px - ix.astype(jnp.float32)
                wy = py - iy.astype(jnp.float32)
                ex = 1.0 - wx
                ey = 1.0 - wy
                d00 = offw + (ix * nl + iy) * 2
                pt = lane + g * 16
                acc0 = jnp.zeros((16,), jnp.float32)
                acc1 = jnp.zeros((16,), jnp.float32)
                for doff, wgt in ((0, ex * ey), (2 * nl, wx * ey),
                                  (2, ex * wy), (2 * nl + 2, wx * wy)):
                    d = d00 + doff
                    acc0 = acc0 + plsc.load_gather(dense, [d]) * wgt
                    acc1 = acc1 + plsc.load_gather(dense, [d + 1]) * wgt
                plsc.store_scatter(encbuf, [pt, f_lo], acc0)
                plsc.store_scatter(encbuf, [pt, f_hi], acc1)

    def it_body(it, _):
        for s in range(R):
            @pl.when(it % R == s)
            def _(s=s):
                @pl.when(it >= R)
                def _():
                    combine(it - R, s, sems[s])

                @pl.when(it < N_IT)
                def _():
                    fire(it, s, sems[s])
        jt = it - R

        @pl.when(jnp.logical_and(jt >= 0, jt % N_STREAM == N_STREAM - 1))
        def _():
            ch = jt // N_STREAM
            coarse_fill(ch)
            pltpu.sync_copy(encbuf, enc.at[pl.ds(base_pt + ch * C, C)])
        return ()

    lax.fori_loop(0, N_IT + R, it_body, ())


_encode = functools.partial(
    pl.kernel,
    mesh=_SC_MESH,
    out_type=jax.ShapeDtypeStruct((N, N_LEVELS * F_PER_LEVEL), jnp.float32),
    compiler_params=_SC_PARAMS,
    scratch_types=[
        pltpu.VMEM((N_W * 2,), jnp.float32),
        pltpu.VMEM((R, C), jnp.float32),
        pltpu.VMEM((R, C), jnp.float32),
        pltpu.VMEM((R, 4, C), jnp.int32),
        pltpu.VMEM((R, 4, C), jnp.int32),
        pltpu.VMEM((R, 4, C, W_ROW), jnp.float32),
        pltpu.VMEM((C, N_LEVELS * F_PER_LEVEL), jnp.float32),
        pltpu.VMEM((N_LEVELS,), jnp.float32),
        pltpu.VMEM((DENSE_TOT * 2,), jnp.float32),
    ] + [pltpu.SemaphoreType.DMA] * R,
)(_encode_body)


BM = 4096


def _mlp_body(enc_ref, w0_ref, w1_ref, w2t_ref, out_ref):
    h = jnp.dot(enc_ref[...], w0_ref[...], preferred_element_type=jnp.float32)
    h = jnp.maximum(h, 0.0)
    h = jnp.dot(h, w1_ref[...], preferred_element_type=jnp.float32)
    h = jnp.maximum(h, 0.0)
    oT = lax.dot_general(w2t_ref[...], h, (((1,), (1,)), ((), ())),
                         preferred_element_type=jnp.float32)
    for j in range(BM // IMG_RES):
        out_ref[:, j, :] = oT[:, j * IMG_RES:(j + 1) * IMG_RES]


def _mlp(enc, W0, W1, W2t):
    n = enc.shape[0]
    rows = n // IMG_RES
    return pl.pallas_call(
        _mlp_body,
        grid=(n // BM,),
        in_specs=[
            pl.BlockSpec((BM, 32), lambda i: (i, 0)),
            pl.BlockSpec((32, 64), lambda i: (0, 0)),
            pl.BlockSpec((64, 64), lambda i: (0, 0)),
            pl.BlockSpec((OUT_FEATURES, 64), lambda i: (0, 0)),
        ],
        out_specs=pl.BlockSpec((OUT_FEATURES, BM // IMG_RES, IMG_RES),
                               lambda i: (0, i, 0)),
        out_shape=jax.ShapeDtypeStruct((OUT_FEATURES, rows, IMG_RES),
                                       jnp.float32),
    )(enc, W0, W1, W2t)


def kernel(xy, tables, W0, W1, W2):
    tabn = (tables.reshape(N_LEVELS, T // 128, 128, F_PER_LEVEL)
            .transpose(0, 1, 3, 2).reshape(TABN_ROWS, 128))
    xyv = xy.reshape(N // 128, 128, 2).transpose(0, 2, 1).reshape(-1)
    tabi = _interleave(tabn).reshape(TAB_WORDS // W_ROW, W_ROW)
    res = jnp.asarray(RES_NP)
    enc = _encode(xyv, tabi, res)
    return _mlp(enc, W0, W1, W2.T)


# speedup vs baseline: 1.0623x; 1.0623x over previous
import functools

import jax
import jax.numpy as jnp
import numpy as np
from jax import lax
from jax.experimental import pallas as pl
from jax.experimental.pallas import tpu as pltpu
from jax.experimental.pallas import tpu_sc as plsc

N_LEVELS = 16
F_PER_LEVEL = 2
LOG2_T = 19
T = 1 << LOG2_T
MASK = T - 1
BASE_RES = 16
PER_LEVEL_SCALE = 1.5
IMG_RES = 512
OUT_FEATURES = 4
N = IMG_RES * IMG_RES

HASH_K = np.int32(-1640531535)

RES_NP = np.array(
    [float(np.floor(BASE_RES * PER_LEVEL_SCALE**l)) for l in range(N_LEVELS)],
    dtype=np.float32,
)

NC = 2
NS = 16
NW = NC * NS
C = 128
G = C // 16
N_W = N // NW
N_CHUNKS = N_W // C

W_ROW = 16
R = 4

N_DENSE = 5
DENSE_NL = [int(RES_NP[l]) + 1 for l in range(N_DENSE)]
DENSE_SZ = [nl * nl for nl in DENSE_NL]
DENSE_OFF = [sum(DENSE_SZ[:l]) for l in range(N_DENSE)]
DENSE_TOT = sum(DENSE_SZ)
DENSE_NB = [(sz + C - 1) // C for sz in DENSE_SZ]
N_STREAM = N_LEVELS - N_DENSE
N_IT = N_CHUNKS * N_STREAM

TAB_WORDS = N_LEVELS * T * F_PER_LEVEL
TABN_ROWS = TAB_WORDS // 128
ROWS_PER_TILE = TABN_ROWS // NW
IL_CHUNK = 64

_SC_PARAMS = pltpu.CompilerParams(
    needs_layout_passes=False, use_tc_tiling_on_sc=False)
_SC_MESH = plsc.VectorSubcoreMesh(core_axis_name="c", subcore_axis_name="s")


IL_N = ROWS_PER_TILE // IL_CHUNK


def _interleave_body(tabn, tabi, inbuf, outbuf, semi, semo):
    wid = lax.axis_index("s") * NC + lax.axis_index("c")
    r0 = wid * ROWS_PER_TILE
    lane = lax.iota(jnp.int32, 16)

    def fire_in(i, s):
        pltpu.async_copy(tabn.at[pl.ds(r0 + i * IL_CHUNK, IL_CHUNK)],
                         inbuf.at[s], semi)

    for s in range(2):
        fire_in(s, s)

    def chunk_body(i, _):
        for s in range(2):
            @pl.when(i % 2 == s)
            def _(s=s):
                pltpu.make_async_copy(tabn.at[pl.ds(0, IL_CHUNK)],
                                      inbuf.at[s], semi).wait()

                @pl.when(i >= 2)
                def _():
                    pltpu.make_async_copy(
                        outbuf.at[s],
                        tabi.at[pl.ds(0, IL_CHUNK * 128)], semo).wait()
                for p in range(IL_CHUNK // 2):
                    for v in range(8):
                        f0 = inbuf[s, 2 * p, pl.ds(v * 16, 16)]
                        f1 = inbuf[s, 2 * p + 1, pl.ds(v * 16, 16)]
                        w = lane * 2 + (p * 256 + v * 32)
                        plsc.store_scatter(outbuf.at[s], [w], f0)
                        plsc.store_scatter(outbuf.at[s], [w + 1], f1)
                rbase = r0 + i * IL_CHUNK
                pltpu.async_copy(
                    outbuf.at[s], tabi.at[pl.ds(rbase * 128, IL_CHUNK * 128)],
                    semo)

                @pl.when(i + 2 < IL_N)
                def _():
                    fire_in(i + 2, s)
        return ()

    lax.fori_loop(0, IL_N, chunk_body, ())
    for s in range(2):
        pltpu.make_async_copy(outbuf.at[s],
                              tabi.at[pl.ds(0, IL_CHUNK * 128)], semo).wait()


_interleave = functools.partial(
    pl.kernel,
    mesh=_SC_MESH,
    out_type=jax.ShapeDtypeStruct((TAB_WORDS,), jnp.float32),
    compiler_params=_SC_PARAMS,
    scratch_types=[
        pltpu.VMEM((2, IL_CHUNK, 128), jnp.float32),
        pltpu.VMEM((2, IL_CHUNK * 128), jnp.float32),
        pltpu.SemaphoreType.DMA,
        pltpu.SemaphoreType.DMA,
    ],
)(_interleave_body)


def _encode_body(xyv, tab, res, enc,
                 xyall, wxbuf, wybuf, idxb, pbuf, rows, encbuf, resb, dense,
                 *sems):
    wid = lax.axis_index("s") * NC + lax.axis_index("c")
    base_pt = wid * N_W
    pltpu.sync_copy(res, resb)
    pltpu.sync_copy(xyv.at[pl.ds(base_pt * 2, N_W * 2)], xyall)
    lane = lax.iota(jnp.int32, 16)
    zeros16 = jnp.zeros((16,), jnp.int32)

    for lvl in range(N_DENSE):
        nl = DENSE_NL[lvl]
        emax = DENSE_SZ[lvl] - 1
        offw = DENSE_OFF[lvl] * 2
        lbase = lvl * T
        nb = DENSE_NB[lvl]

        def fire_b(b, s, nl=nl, emax=emax, lbase=lbase):
            for g in range(G):
                sl = pl.ds(g * 16, 16)
                e = jnp.minimum(lane + (b * C + g * 16), emax)
                cx = e // nl
                cy = e - cx * nl
                h = (cx ^ (cy * HASH_K)) & MASK
                idxb[s, 0, sl] = (lbase + h) >> 3
                pbuf[s, 0, sl] = (h & 7) * 2
            pltpu.async_copy(tab.at[idxb.at[s, 0]], rows.at[s, 0], sems[s])

        def extract_b(b, s, nl=nl, emax=emax, offw=offw):
            pltpu.make_async_copy(tab.at[pl.ds(0, C)], rows.at[s, 0],
                                  sems[s]).wait()
            sf = jnp.full((16,), s, jnp.int32)
            for g in range(G):
                sl = pl.ds(g * 16, 16)
                e = jnp.minimum(lane + (b * C + g * 16), emax)
                d = offw + e * 2
                p = pbuf[s, 0, sl]
                pt = lane + g * 16
                f0 = plsc.load_gather(rows, [sf, zeros16, pt, p])
                f1 = plsc.load_gather(rows, [sf, zeros16, pt, p + 1])
                plsc.store_scatter(dense, [d], f0)
                plsc.store_scatter(dense, [d + 1], f1)

        def build_body(b, _, fire_b=fire_b, extract_b=extract_b, nb=nb):
            for s in range(2):
                @pl.when(b % 2 == s)
                def _(s=s):
                    @pl.when(b >= 2)
                    def _():
                        extract_b(b - 2, s)

                    @pl.when(b < nb)
                    def _():
                        fire_b(b, s)
            return ()

        lax.fori_loop(0, nb + 2, build_body, ())

    def fire(it, s, sem):
        ch = it // N_STREAM
        l = N_DENSE + it % N_STREAM
        resv = plsc.load_gather(resb, [jnp.full((16,), l, jnp.int32)])
        lbase = l * T
        for g in range(G):
            sl = pl.ds(g * 16, 16)
            px = xyall[pl.ds(ch * 256 + g * 16, 16)] * resv
            py = xyall[pl.ds(ch * 256 + 128 + g * 16, 16)] * resv
            ix = px.astype(jnp.int32)
            iy = py.astype(jnp.int32)
            wxbuf[s, sl] = px - ix.astype(jnp.float32)
            wybuf[s, sl] = py - iy.astype(jnp.float32)
            hy0 = iy * HASH_K
            hy1 = (iy + 1) * HASH_K
            ix1 = ix + 1
            for c, h in ((0, (ix ^ hy0) & MASK), (1, (ix1 ^ hy0) & MASK),
                         (2, (ix ^ hy1) & MASK), (3, (ix1 ^ hy1) & MASK)):
                idxb[s, c, sl] = (lbase + h) >> 3
                pbuf[s, c, sl] = (h & 7) * 2
        for c in range(4):
            pltpu.async_copy(tab.at[idxb.at[s, c]], rows.at[s, c], sem)

    def combine(it, s, sem):
        l = N_DENSE + it % N_STREAM
        for c in range(4):
            pltpu.make_async_copy(tab.at[pl.ds(0, C)], rows.at[s, c],
                                  sem).wait()
        sf = jnp.full((16,), s, jnp.int32)
        f_lo = jnp.full((16,), 2 * l, jnp.int32)
        f_hi = f_lo + 1
        for g in range(G):
            sl = pl.ds(g * 16, 16)
            wx = wxbuf[s, sl]
            wy = wybuf[s, sl]
            ex = 1.0 - wx
            ey = 1.0 - wy
            pt = lane + g * 16
            acc0 = jnp.zeros((16,), jnp.float32)
            acc1 = jnp.zeros((16,), jnp.float32)
            for c, wgt in ((0, ex * ey), (1, wx * ey),
                           (2, ex * wy), (3, wx * wy)):
                cf = jnp.full((16,), c, jnp.int32)
                p = pbuf[s, c, sl]
                f0 = plsc.load_gather(rows, [sf, cf, pt, p])
                f1 = plsc.load_gather(rows, [sf, cf, pt, p + 1])
                acc0 = acc0 + f0 * wgt
                acc1 = acc1 + f1 * wgt
            plsc.store_scatter(encbuf, [pt, f_lo], acc0)
            plsc.store_scatter(encbuf, [pt, f_hi], acc1)

    def coarse_fill(ch):
        for lvl in range(N_DENSE):
            nl = DENSE_NL[lvl]
            offw = DENSE_OFF[lvl] * 2
            resv = jnp.full((16,), float(RES_NP[lvl]), jnp.float32)
            f_lo = jnp.full((16,), 2 * lvl, jnp.int32)
            f_hi = f_lo + 1
            for g in range(G):
                px = xyall[pl.ds(ch * 256 + g * 16, 16)] * resv
                py = xyall[pl.ds(ch * 256 + 128 + g * 16, 16)] * resv
                ix = px.astype(jnp.int32)
                iy = py.astype(jnp.int32)
                wx = px - ix.astype(jnp.float32)
                wy = py - iy.astype(jnp.float32)
                ex = 1.0 - wx
                ey = 1.0 - wy
                d00 = offw + (ix * nl + iy) * 2
                pt = lane + g * 16
                acc0 = jnp.zeros((16,), jnp.float32)
                acc1 = jnp.zeros((16,), jnp.float32)
                for doff, wgt in ((0, ex * ey), (2 * nl, wx * ey),
                                  (2, ex * wy), (2 * nl + 2, wx * wy)):
                    d = d00 + doff
                    acc0 = acc0 + plsc.load_gather(dense, [d]) * wgt
                    acc1 = acc1 + plsc.load_gather(dense, [d + 1]) * wgt
                plsc.store_scatter(encbuf, [pt, f_lo], acc0)
                plsc.store_scatter(encbuf, [pt, f_hi], acc1)

    def it_body(it, _):
        for s in range(R):
            @pl.when(it % R == s)
            def _(s=s):
                @pl.when(it >= R)
                def _():
                    combine(it - R, s, sems[s])

                @pl.when(it < N_IT)
                def _():
                    fire(it, s, sems[s])
        jt = it - R

        @pl.when(jnp.logical_and(jt >= 0, jt % N_STREAM == N_STREAM - 1))
        def _():
            ch = jt // N_STREAM
            coarse_fill(ch)
            pltpu.sync_copy(encbuf, enc.at[pl.ds(base_pt + ch * C, C)])
        return ()

    lax.fori_loop(0, N_IT + R, it_body, ())


_encode = functools.partial(
    pl.kernel,
    mesh=_SC_MESH,
    out_type=jax.ShapeDtypeStruct((N, N_LEVELS * F_PER_LEVEL), jnp.float32),
    compiler_params=_SC_PARAMS,
    scratch_types=[
        pltpu.VMEM((N_W * 2,), jnp.float32),
        pltpu.VMEM((R, C), jnp.float32),
        pltpu.VMEM((R, C), jnp.float32),
        pltpu.VMEM((R, 4, C), jnp.int32),
        pltpu.VMEM((R, 4, C), jnp.int32),
        pltpu.VMEM((R, 4, C, W_ROW), jnp.float32),
        pltpu.VMEM((C, N_LEVELS * F_PER_LEVEL), jnp.float32),
        pltpu.VMEM((N_LEVELS,), jnp.float32),
        pltpu.VMEM((DENSE_TOT * 2,), jnp.float32),
    ] + [pltpu.SemaphoreType.DMA] * R,
)(_encode_body)


BM = 4096


def _mlp_body(enc_ref, w0_ref, w1_ref, w2t_ref, out_ref):
    h = jnp.dot(enc_ref[...], w0_ref[...], preferred_element_type=jnp.float32)
    h = jnp.maximum(h, 0.0)
    h = jnp.dot(h, w1_ref[...], preferred_element_type=jnp.float32)
    h = jnp.maximum(h, 0.0)
    oT = lax.dot_general(w2t_ref[...], h, (((1,), (1,)), ((), ())),
                         preferred_element_type=jnp.float32)
    for j in range(BM // IMG_RES):
        out_ref[:, j, :] = oT[:, j * IMG_RES:(j + 1) * IMG_RES]


def _mlp(enc, W0, W1, W2t):
    n = enc.shape[0]
    rows = n // IMG_RES
    return pl.pallas_call(
        _mlp_body,
        grid=(n // BM,),
        in_specs=[
            pl.BlockSpec((BM, 32), lambda i: (i, 0)),
            pl.BlockSpec((32, 64), lambda i: (0, 0)),
            pl.BlockSpec((64, 64), lambda i: (0, 0)),
            pl.BlockSpec((OUT_FEATURES, 64), lambda i: (0, 0)),
        ],
        out_specs=pl.BlockSpec((OUT_FEATURES, BM // IMG_RES, IMG_RES),
                               lambda i: (0, i, 0)),
        out_shape=jax.ShapeDtypeStruct((OUT_FEATURES, rows, IMG_RES),
                                       jnp.float32),
    )(enc, W0, W1, W2t)


def kernel(xy, tables, W0, W1, W2):
    tabn = (tables.reshape(N_LEVELS, T // 128, 128, F_PER_LEVEL)
            .transpose(0, 1, 3, 2).reshape(TABN_ROWS, 128))
    xyv = xy.reshape(N // 128, 128, 2).transpose(0, 2, 1).reshape(-1)
    tabi = _interleave(tabn).reshape(TAB_WORDS // W_ROW, W_ROW)
    res = jnp.asarray(RES_NP)
    enc = _encode(xyv, tabi, res)
    return _mlp(enc, W0, W1, W2.T)
